# baseline (speedup 1.0000x reference)
import jax
import jax.numpy as jnp
from jax import lax
from jax.experimental import pallas as pl
from jax.experimental.pallas import tpu as pltpu
from jax.experimental.pallas import tpu_sc as plsc

B = 32768
D = 2048
NC = 2
NS = 16
NW = NC * NS
BPW = B // NW
C = 16
NCHUNK = BPW // C


NB = 3


def _emb_body(idx_hbm, table_hbm, out_hbm, idx_v, buf0, buf1, buf2,
              gsem0, gsem1, gsem2, osem0, osem1, osem2):
    wid = lax.axis_index("s") * NC + lax.axis_index("c")
    base = wid * BPW
    pltpu.sync_copy(idx_hbm.at[pl.ds(base, BPW)], idx_v)

    bufs = (buf0, buf1, buf2)
    gsems = (gsem0, gsem1, gsem2)
    osems = (osem0, osem1, osem2)

    def gather(g):
        return pltpu.make_async_copy(
            table_hbm.at[idx_v.at[pl.ds(g * C, C)]], bufs[g % NB],
            gsems[g % NB])

    def store(g):
        return pltpu.make_async_copy(
            bufs[g % NB], out_hbm.at[pl.ds(base + g * C, C)], osems[g % NB])

    gather(0).start()
    gather(1).start()
    for g in range(NCHUNK):
        gather(g).wait()
        store(g).start()
        if g + 2 < NCHUNK:
            if g >= 1:
                store(g - 1).wait()
            gather(g + 2).start()
    store(NCHUNK - 3).wait()
    store(NCHUNK - 2).wait()
    store(NCHUNK - 1).wait()


_emb = pl.kernel(
    _emb_body,
    out_type=jax.ShapeDtypeStruct((B, D), jnp.float32),
    mesh=plsc.VectorSubcoreMesh(core_axis_name="c", subcore_axis_name="s"),
    scratch_types=[
        pltpu.VMEM((BPW,), jnp.int32),
        pltpu.VMEM((C, D), jnp.float32),
        pltpu.VMEM((C, D), jnp.float32),
        pltpu.VMEM((C, D), jnp.float32),
        pltpu.SemaphoreType.DMA,
        pltpu.SemaphoreType.DMA,
        pltpu.SemaphoreType.DMA,
        pltpu.SemaphoreType.DMA,
        pltpu.SemaphoreType.DMA,
        pltpu.SemaphoreType.DMA,
    ],
)


def kernel(position_ids, weight):
    idx = position_ids.reshape(-1).astype(jnp.int32)
    out = _emb(idx, weight)
    return out.reshape(position_ids.shape + (weight.shape[1],))


# speedup vs baseline: 1.0050x; 1.0050x over previous
import jax
import jax.numpy as jnp
from jax import lax
from jax.experimental import pallas as pl
from jax.experimental.pallas import tpu as pltpu
from jax.experimental.pallas import tpu_sc as plsc

B = 32768
D = 2048
NC = 2
NS = 16
NW = NC * NS
BPW = B // NW
C = 16
NCHUNK = BPW // C


def _emb_body(idx_hbm, table_hbm, out_hbm, idx_v, buf0, buf1,
              gsem0, gsem1, osem0, osem1):
    wid = lax.axis_index("s") * NC + lax.axis_index("c")
    base = wid * BPW
    pltpu.sync_copy(idx_hbm.at[pl.ds(base, BPW)], idx_v)

    bufs = (buf0, buf1)
    gsems = (gsem0, gsem1)
    osems = (osem0, osem1)

    def gather(g):
        return pltpu.make_async_copy(
            table_hbm.at[idx_v.at[pl.ds(g * C, C)]], bufs[g % 2],
            gsems[g % 2])

    def store(g):
        return pltpu.make_async_copy(
            bufs[g % 2], out_hbm.at[pl.ds(base + g * C, C)], osems[g % 2])

    gather(0).start()
    gather(1).start()
    for g in range(NCHUNK):
        gather(g).wait()
        store(g).start()
        store(g).wait()
        if g + 2 < NCHUNK:
            gather(g + 2).start()


_emb = pl.kernel(
    _emb_body,
    out_type=jax.ShapeDtypeStruct((B, D), jnp.float32),
    mesh=plsc.VectorSubcoreMesh(core_axis_name="c", subcore_axis_name="s"),
    scratch_types=[
        pltpu.VMEM((BPW,), jnp.int32),
        pltpu.VMEM((C, D), jnp.float32),
        pltpu.VMEM((C, D), jnp.float32),
        pltpu.SemaphoreType.DMA,
        pltpu.SemaphoreType.DMA,
        pltpu.SemaphoreType.DMA,
        pltpu.SemaphoreType.DMA,
    ],
)


def kernel(position_ids, weight):
    idx = position_ids.reshape(-1).astype(jnp.int32)
    out = _emb(idx, weight)
    return out.reshape(position_ids.shape + (weight.shape[1],))


# speedup vs baseline: 1.0366x; 1.0314x over previous
import jax
import jax.numpy as jnp
from jax import lax
from jax.experimental import pallas as pl
from jax.experimental.pallas import tpu as pltpu
from jax.experimental.pallas import tpu_sc as plsc

B = 32768
D = 2048
NC = 2
NS = 16
NW = NC * NS
BPW = B // NW
C = 16
NCHUNK = BPW // C


def _emb_body(idx_hbm, table_hbm, out_hbm, idx_v, buf0, buf1,
              gsem0, gsem1, osem0, osem1):
    wid = lax.axis_index("s") * NC + lax.axis_index("c")
    base = wid * BPW
    pltpu.sync_copy(idx_hbm.at[pl.ds(base, BPW)], idx_v)

    bufs = (buf0, buf1)
    gsems = (gsem0, gsem1)
    osems = (osem0, osem1)

    def gather(g, b):
        return pltpu.make_async_copy(
            table_hbm.at[idx_v.at[pl.ds(g * C, C)]], bufs[b], gsems[b])

    def store(g, b):
        return pltpu.make_async_copy(
            bufs[b], out_hbm.at[pl.ds(base + g * C, C)], osems[b])

    gather(0, 0).start()
    gather(1, 1).start()

    def step(h, last):
        for b in range(2):
            g = 2 * h + b
            gather(g, b).wait()
            store(g, b).start()
            store(g, b).wait()
            if not last:
                gather(g + 2, b).start()

    def body(h, carry):
        step(h, last=False)
        return carry

    lax.fori_loop(0, NCHUNK // 2 - 1, body, 0)
    step(NCHUNK // 2 - 1, last=True)


_emb = pl.kernel(
    _emb_body,
    out_type=jax.ShapeDtypeStruct((B, D), jnp.float32),
    mesh=plsc.VectorSubcoreMesh(core_axis_name="c", subcore_axis_name="s"),
    scratch_types=[
        pltpu.VMEM((BPW,), jnp.int32),
        pltpu.VMEM((C, D), jnp.float32),
        pltpu.VMEM((C, D), jnp.float32),
        pltpu.SemaphoreType.DMA,
        pltpu.SemaphoreType.DMA,
        pltpu.SemaphoreType.DMA,
        pltpu.SemaphoreType.DMA,
    ],
)


def kernel(position_ids, weight):
    idx = position_ids.reshape(-1).astype(jnp.int32)
    out = _emb(idx, weight)
    return out.reshape(position_ids.shape + (weight.shape[1],))
